# baseline (speedup 1.0000x reference)
import functools

import jax
import jax.numpy as jnp
from jax import lax
from jax.experimental import pallas as pl
from jax.experimental.pallas import tpu as pltpu
from jax.experimental.pallas import tpu_sc as plsc

N_NODES_ = 10000
N_EDGES_ = 320000
D_ = 128

NC = 2
NS = 16
WPT = 4

E_SC = N_EDGES_ // NC
CE = 3200
N_CE = E_SC // CE
COLS = 320
N_RED = CE // COLS


def _decode_mul_add(acc, aw, bw):
    a_hi = lax.bitcast_convert_type(aw, jnp.float32)
    b_hi = lax.bitcast_convert_type(bw, jnp.float32)
    a_lo = lax.bitcast_convert_type(aw << 16, jnp.float32)
    b_lo = lax.bitcast_convert_type(bw << 16, jnp.float32)
    return acc + (a_hi * b_hi + a_lo * b_lo)


def _edge_decoder_kernel(tab_a_hbm, tab_b_hbm, idx_src_hbm, idx_dst_hbm,
                         out_hbm, parts_hbm, loc_a, loc_b, is0, id0, is1, id1,
                         pacc, rbuf, obuf, sem_i0, sem_i1, sem_r):
    sid = lax.axis_index("s")
    c = lax.axis_index("c")
    base_sc = c * E_SC

    pltpu.sync_copy(tab_a_hbm.at[sid], loc_a)
    pltpu.sync_copy(tab_b_hbm.at[sid], loc_b)

    kvecs = [jnp.full((16,), k, jnp.int32) for k in range(WPT)]

    def compute_chunk(isb, idb):
        @plsc.parallel_loop(0, CE // 16, 1, unroll=2)
        def _(g):
            svw = isb[pl.ds(g * 16, 16)] << 2
            dvw = idb[pl.ds(g * 16, 16)] << 2
            acc = jnp.zeros((16,), jnp.float32)
            for k in range(WPT):
                aw = plsc.load_gather(loc_a, [svw + kvecs[k]])
                bw = plsc.load_gather(loc_b, [dvw + kvecs[k]])
                acc = _decode_mul_add(acc, aw, bw)
            pacc[pl.ds(g * 16, 16)] = acc

    def stream_parts(ch):
        pltpu.sync_copy(pacc, parts_hbm.at[c, sid, pl.ds(ch * CE, CE)])

    def reduce_chunk(ch):
        @pl.when(sid < N_RED)
        def _():
            col0 = ch * CE + sid * COLS
            for r in range(NS):
                pltpu.async_copy(
                    parts_hbm.at[c, r, pl.ds(col0, COLS)], rbuf.at[r], sem_r)
            for r in range(NS):
                pltpu.make_async_copy(
                    parts_hbm.at[c, r, pl.ds(col0, COLS)], rbuf.at[r],
                    sem_r).wait()
            for j in range(COLS // 16):
                s = rbuf[0, pl.ds(j * 16, 16)]
                for r in range(1, NS):
                    s = s + rbuf[r, pl.ds(j * 16, 16)]
                obuf[pl.ds(j * 16, 16)] = s
            pltpu.sync_copy(obuf, out_hbm.at[pl.ds(base_sc + col0, COLS)])

    def issue_idx(ch, isb, idb, sem):
        off = base_sc + ch * CE
        pltpu.async_copy(idx_src_hbm.at[pl.ds(off, CE)], isb, sem)
        pltpu.async_copy(idx_dst_hbm.at[pl.ds(off, CE)], idb, sem)

    def wait_idx(isb, idb, sem):
        pltpu.make_async_copy(idx_src_hbm.at[pl.ds(0, CE)], isb, sem).wait()
        pltpu.make_async_copy(idx_dst_hbm.at[pl.ds(0, CE)], idb, sem).wait()

    pltpu.sync_copy(idx_src_hbm.at[pl.ds(base_sc, CE)], is0)
    pltpu.sync_copy(idx_dst_hbm.at[pl.ds(base_sc, CE)], id0)
    issue_idx(1, is1, id1, sem_i1)

    def pair_body(g, _):
        c0 = 2 * g
        compute_chunk(is0, id0)
        stream_parts(c0)

        @pl.when(g < N_CE // 2 - 1)
        def _():
            issue_idx(c0 + 2, is0, id0, sem_i0)

        plsc.subcore_barrier()
        reduce_chunk(c0)

        wait_idx(is1, id1, sem_i1)
        compute_chunk(is1, id1)
        stream_parts(c0 + 1)

        @pl.when(g < N_CE // 2 - 1)
        def _():
            issue_idx(c0 + 3, is1, id1, sem_i1)

        plsc.subcore_barrier()
        reduce_chunk(c0 + 1)

        @pl.when(g < N_CE // 2 - 1)
        def _():
            wait_idx(is0, id0, sem_i0)

        return 0

    lax.fori_loop(0, N_CE // 2, pair_body, 0)


@jax.jit
def _edge_decoder(tab_a, tab_b, idx_src, idx_dst):
    mesh = plsc.VectorSubcoreMesh(core_axis_name="c", subcore_axis_name="s")
    kfn = functools.partial(
        pl.kernel,
        mesh=mesh,
        out_type=[
            jax.ShapeDtypeStruct((N_EDGES_,), jnp.float32),
            jax.ShapeDtypeStruct((NC, NS, E_SC), jnp.float32),
        ],
        compiler_params=pltpu.CompilerParams(
            use_tc_tiling_on_sc=False, needs_layout_passes=False),
        scratch_types=[
            pltpu.VMEM((N_NODES_ * WPT,), jnp.int32),
            pltpu.VMEM((N_NODES_ * WPT,), jnp.int32),
            pltpu.VMEM((CE,), jnp.int32),
            pltpu.VMEM((CE,), jnp.int32),
            pltpu.VMEM((CE,), jnp.int32),
            pltpu.VMEM((CE,), jnp.int32),
            pltpu.VMEM((CE,), jnp.float32),
            pltpu.VMEM((NS, COLS), jnp.float32),
            pltpu.VMEM((COLS,), jnp.float32),
            pltpu.SemaphoreType.DMA,
            pltpu.SemaphoreType.DMA,
            pltpu.SemaphoreType.DMA,
        ],
    )(_edge_decoder_kernel)
    return kfn(tab_a, tab_b, idx_src, idx_dst)[0]


def _pack_table(x):
    xb = x.astype(jnp.bfloat16).reshape(N_NODES_, NS, WPT, 2)
    xt = xb.transpose(1, 0, 2, 3)
    return jax.lax.bitcast_convert_type(xt, jnp.int32).reshape(
        NS, N_NODES_ * WPT)


def kernel(x_src, x_dst, edge_label_index):
    idx = edge_label_index.astype(jnp.int32)
    return _edge_decoder(_pack_table(x_src), _pack_table(x_dst),
                         idx[0], idx[1])


# speedup vs baseline: 1.2097x; 1.2097x over previous
import functools

import jax
import jax.numpy as jnp
from jax import lax
from jax.experimental import pallas as pl
from jax.experimental.pallas import tpu as pltpu
from jax.experimental.pallas import tpu_sc as plsc

N_NODES_ = 10000
N_EDGES_ = 320000
D_ = 128

NC = 2
NS = 16
WPT = 4

E_SC = N_EDGES_ // NC
CE = 3200
N_CE = E_SC // CE
BLK = 128
N_BLK = CE // BLK


def _decode_mul_add(acc, aw, bw):
    a_hi = lax.bitcast_convert_type(aw, jnp.float32)
    b_hi = lax.bitcast_convert_type(bw, jnp.float32)
    a_lo = lax.bitcast_convert_type(aw << 16, jnp.float32)
    b_lo = lax.bitcast_convert_type(bw << 16, jnp.float32)
    return acc + (a_hi * b_hi + a_lo * b_lo)


def _edge_decoder_kernel(tab_a_hbm, tab_b_hbm, idx_src_hbm, idx_dst_hbm,
                         out_hbm, loc_a, loc_b, is0, id0, is1, id1,
                         pacc, rbuf, obuf, parts_sp, sem_i0, sem_i1, sem_r):
    sid = lax.axis_index("s")
    c = lax.axis_index("c")
    base_sc = c * E_SC

    pltpu.sync_copy(tab_a_hbm.at[sid], loc_a)
    pltpu.sync_copy(tab_b_hbm.at[sid], loc_b)

    kvecs = [jnp.full((16,), k, jnp.int32) for k in range(WPT)]

    def compute_chunk(isb, idb):
        @plsc.parallel_loop(0, CE // 16, 1, unroll=2)
        def _(g):
            svw = isb[pl.ds(g * 16, 16)] << 2
            dvw = idb[pl.ds(g * 16, 16)] << 2
            acc = jnp.zeros((16,), jnp.float32)
            for k in range(WPT):
                aw = plsc.load_gather(loc_a, [svw + kvecs[k]])
                bw = plsc.load_gather(loc_b, [dvw + kvecs[k]])
                acc = _decode_mul_add(acc, aw, bw)
            pacc[pl.ds(g * 16, 16)] = acc

    def stream_parts(buf):
        pltpu.sync_copy(pacc, parts_sp.at[sid, pl.ds(buf * CE, CE)])

    def reduce_chunk(ch, buf):
        for it in range(2):
            blk = sid + 16 * it

            @pl.when(blk < N_BLK)
            def _():
                off = buf * CE + blk * BLK
                for r in range(NS):
                    pltpu.async_copy(
                        parts_sp.at[r, pl.ds(off, BLK)], rbuf.at[r], sem_r)
                for r in range(NS):
                    pltpu.make_async_copy(
                        parts_sp.at[r, pl.ds(off, BLK)], rbuf.at[r],
                        sem_r).wait()
                for j in range(BLK // 16):
                    s = rbuf[0, pl.ds(j * 16, 16)]
                    for r in range(1, NS):
                        s = s + rbuf[r, pl.ds(j * 16, 16)]
                    obuf[pl.ds(j * 16, 16)] = s
                pltpu.sync_copy(
                    obuf,
                    out_hbm.at[pl.ds(base_sc + ch * CE + blk * BLK, BLK)])

    def issue_idx(ch, isb, idb, sem):
        off = base_sc + ch * CE
        pltpu.async_copy(idx_src_hbm.at[pl.ds(off, CE)], isb, sem)
        pltpu.async_copy(idx_dst_hbm.at[pl.ds(off, CE)], idb, sem)

    def wait_idx(isb, idb, sem):
        pltpu.make_async_copy(idx_src_hbm.at[pl.ds(0, CE)], isb, sem).wait()
        pltpu.make_async_copy(idx_dst_hbm.at[pl.ds(0, CE)], idb, sem).wait()

    pltpu.sync_copy(idx_src_hbm.at[pl.ds(base_sc, CE)], is0)
    pltpu.sync_copy(idx_dst_hbm.at[pl.ds(base_sc, CE)], id0)
    issue_idx(1, is1, id1, sem_i1)

    def pair_body(g, _):
        c0 = 2 * g
        compute_chunk(is0, id0)
        stream_parts(0)

        @pl.when(g < N_CE // 2 - 1)
        def _():
            issue_idx(c0 + 2, is0, id0, sem_i0)

        plsc.subcore_barrier()
        reduce_chunk(c0, 0)

        wait_idx(is1, id1, sem_i1)
        compute_chunk(is1, id1)
        stream_parts(1)

        @pl.when(g < N_CE // 2 - 1)
        def _():
            issue_idx(c0 + 3, is1, id1, sem_i1)

        plsc.subcore_barrier()
        reduce_chunk(c0 + 1, 1)

        @pl.when(g < N_CE // 2 - 1)
        def _():
            wait_idx(is0, id0, sem_i0)

        return 0

    lax.fori_loop(0, N_CE // 2, pair_body, 0)


@jax.jit
def _edge_decoder(tab_a, tab_b, idx_src, idx_dst):
    mesh = plsc.VectorSubcoreMesh(core_axis_name="c", subcore_axis_name="s")
    kfn = functools.partial(
        pl.kernel,
        mesh=mesh,
        out_type=jax.ShapeDtypeStruct((N_EDGES_,), jnp.float32),
        compiler_params=pltpu.CompilerParams(
            use_tc_tiling_on_sc=False, needs_layout_passes=False),
        scratch_types=[
            pltpu.VMEM((N_NODES_ * WPT,), jnp.int32),
            pltpu.VMEM((N_NODES_ * WPT,), jnp.int32),
            pltpu.VMEM((CE,), jnp.int32),
            pltpu.VMEM((CE,), jnp.int32),
            pltpu.VMEM((CE,), jnp.int32),
            pltpu.VMEM((CE,), jnp.int32),
            pltpu.VMEM((CE,), jnp.float32),
            pltpu.VMEM((NS, BLK), jnp.float32),
            pltpu.VMEM((BLK,), jnp.float32),
            pltpu.VMEM_SHARED((NS, 2 * CE), jnp.float32),
            pltpu.SemaphoreType.DMA,
            pltpu.SemaphoreType.DMA,
            pltpu.SemaphoreType.DMA,
        ],
    )(_edge_decoder_kernel)
    return kfn(tab_a, tab_b, idx_src, idx_dst)


def _pack_table(x):
    xb = x.astype(jnp.bfloat16).reshape(N_NODES_, NS, WPT, 2)
    xt = xb.transpose(1, 0, 2, 3)
    return jax.lax.bitcast_convert_type(xt, jnp.int32).reshape(
        NS, N_NODES_ * WPT)


def kernel(x_src, x_dst, edge_label_index):
    idx = edge_label_index.astype(jnp.int32)
    return _edge_decoder(_pack_table(x_src), _pack_table(x_dst),
                         idx[0], idx[1])


# speedup vs baseline: 1.2749x; 1.0539x over previous
import functools

import jax
import jax.numpy as jnp
from jax import lax
from jax.experimental import pallas as pl
from jax.experimental.pallas import tpu as pltpu
from jax.experimental.pallas import tpu_sc as plsc

N_NODES_ = 10000
N_EDGES_ = 320000
D_ = 128

NC = 2
NS = 16
WPT = 4

E_SC = N_EDGES_ // NC
CE = 3200
N_CE = E_SC // CE
BLK = 128
N_BLK = CE // BLK


def _decode_mul_add(acc, aw, bw):
    a_hi = lax.bitcast_convert_type(aw, jnp.float32)
    b_hi = lax.bitcast_convert_type(bw, jnp.float32)
    a_lo = lax.bitcast_convert_type(aw << 16, jnp.float32)
    b_lo = lax.bitcast_convert_type(bw << 16, jnp.float32)
    return acc + (a_hi * b_hi + a_lo * b_lo)


def _edge_decoder_kernel(tab_a_hbm, tab_b_hbm, idx_src_hbm, idx_dst_hbm,
                         out_hbm, loc_a, loc_b, is0, id0, is1, id1,
                         pacc, rbuf, obuf, parts_sp, sem_i0, sem_i1, sem_r,
                         sem_o):
    sid = lax.axis_index("s")
    c = lax.axis_index("c")
    base_sc = c * E_SC

    pltpu.sync_copy(tab_a_hbm.at[sid], loc_a)
    pltpu.sync_copy(tab_b_hbm.at[sid], loc_b)

    kvecs = [jnp.full((16,), k, jnp.int32) for k in range(WPT)]

    def compute_chunk(isb, idb):
        @plsc.parallel_loop(0, CE // 16, 1, unroll=2)
        def _(g):
            svw = isb[pl.ds(g * 16, 16)] << 2
            dvw = idb[pl.ds(g * 16, 16)] << 2
            acc = jnp.zeros((16,), jnp.float32)
            for k in range(WPT):
                aw = plsc.load_gather(loc_a, [svw + kvecs[k]])
                bw = plsc.load_gather(loc_b, [dvw + kvecs[k]])
                acc = _decode_mul_add(acc, aw, bw)
            pacc[pl.ds(g * 16, 16)] = acc

    def stream_parts(buf):
        pltpu.sync_copy(pacc, parts_sp.at[sid, pl.ds(buf * CE, CE)])

    def issue_reduce(buf):
        for it in range(2):
            blk = sid + 16 * it

            @pl.when(blk < N_BLK)
            def _():
                off = buf * CE + blk * BLK
                for r in range(NS):
                    pltpu.async_copy(parts_sp.at[r, pl.ds(off, BLK)],
                                     rbuf.at[it, r], sem_r)

    def finish_reduce(ch, buf, drain_prev):
        for it in range(2):
            blk = sid + 16 * it

            @pl.when(blk < N_BLK)
            def _():
                off = buf * CE + blk * BLK
                dst = out_hbm.at[pl.ds(base_sc + ch * CE + blk * BLK, BLK)]
                for r in range(NS):
                    pltpu.make_async_copy(parts_sp.at[r, pl.ds(off, BLK)],
                                          rbuf.at[it, r], sem_r).wait()

                @pl.when(drain_prev)
                def _():
                    pltpu.make_async_copy(obuf.at[buf, it], dst, sem_o).wait()

                for j in range(BLK // 16):
                    s = rbuf[it, 0, pl.ds(j * 16, 16)]
                    for r in range(1, NS):
                        s = s + rbuf[it, r, pl.ds(j * 16, 16)]
                    obuf[buf, it, pl.ds(j * 16, 16)] = s
                pltpu.async_copy(obuf.at[buf, it], dst, sem_o)

    def last_drain(buf):
        for it in range(2):
            blk = sid + 16 * it

            @pl.when(blk < N_BLK)
            def _():
                dst = out_hbm.at[pl.ds(blk * BLK, BLK)]
                pltpu.make_async_copy(obuf.at[buf, it], dst, sem_o).wait()

    def issue_idx(ch, isb, idb, sem):
        off = base_sc + ch * CE
        pltpu.async_copy(idx_src_hbm.at[pl.ds(off, CE)], isb, sem)
        pltpu.async_copy(idx_dst_hbm.at[pl.ds(off, CE)], idb, sem)

    def wait_idx(isb, idb, sem):
        pltpu.make_async_copy(idx_src_hbm.at[pl.ds(0, CE)], isb, sem).wait()
        pltpu.make_async_copy(idx_dst_hbm.at[pl.ds(0, CE)], idb, sem).wait()

    pltpu.sync_copy(idx_src_hbm.at[pl.ds(base_sc, CE)], is0)
    pltpu.sync_copy(idx_dst_hbm.at[pl.ds(base_sc, CE)], id0)
    issue_idx(1, is1, id1, sem_i1)

    def pair_body(g, _):
        c0 = 2 * g
        @pl.when(g > 0)
        def _():
            issue_reduce(1)

        compute_chunk(is0, id0)

        @pl.when(g > 0)
        def _():
            finish_reduce(c0 - 1, 1, g > 1)

        stream_parts(0)

        @pl.when(g < N_CE // 2 - 1)
        def _():
            issue_idx(c0 + 2, is0, id0, sem_i0)

        plsc.subcore_barrier()
        issue_reduce(0)

        wait_idx(is1, id1, sem_i1)
        compute_chunk(is1, id1)
        finish_reduce(c0, 0, g > 0)
        stream_parts(1)

        @pl.when(g < N_CE // 2 - 1)
        def _():
            issue_idx(c0 + 3, is1, id1, sem_i1)

        plsc.subcore_barrier()

        @pl.when(g < N_CE // 2 - 1)
        def _():
            wait_idx(is0, id0, sem_i0)

        return 0

    lax.fori_loop(0, N_CE // 2, pair_body, 0)

    issue_reduce(1)
    finish_reduce(N_CE - 1, 1, N_CE > 3)
    last_drain(0)
    last_drain(1)


@jax.jit
def _edge_decoder(tab_a, tab_b, idx_src, idx_dst):
    mesh = plsc.VectorSubcoreMesh(core_axis_name="c", subcore_axis_name="s")
    kfn = functools.partial(
        pl.kernel,
        mesh=mesh,
        out_type=jax.ShapeDtypeStruct((N_EDGES_,), jnp.float32),
        compiler_params=pltpu.CompilerParams(
            use_tc_tiling_on_sc=False, needs_layout_passes=False),
        scratch_types=[
            pltpu.VMEM((N_NODES_ * WPT,), jnp.int32),
            pltpu.VMEM((N_NODES_ * WPT,), jnp.int32),
            pltpu.VMEM((CE,), jnp.int32),
            pltpu.VMEM((CE,), jnp.int32),
            pltpu.VMEM((CE,), jnp.int32),
            pltpu.VMEM((CE,), jnp.int32),
            pltpu.VMEM((CE,), jnp.float32),
            pltpu.VMEM((2, NS, BLK), jnp.float32),
            pltpu.VMEM((2, 2, BLK), jnp.float32),
            pltpu.VMEM_SHARED((NS, 2 * CE), jnp.float32),
            pltpu.SemaphoreType.DMA,
            pltpu.SemaphoreType.DMA,
            pltpu.SemaphoreType.DMA,
            pltpu.SemaphoreType.DMA,
        ],
    )(_edge_decoder_kernel)
    return kfn(tab_a, tab_b, idx_src, idx_dst)


def _pack_table(x):
    xb = x.astype(jnp.bfloat16).reshape(N_NODES_, NS, WPT, 2)
    xt = xb.transpose(1, 0, 2, 3)
    return jax.lax.bitcast_convert_type(xt, jnp.int32).reshape(
        NS, N_NODES_ * WPT)


def kernel(x_src, x_dst, edge_label_index):
    idx = edge_label_index.astype(jnp.int32)
    return _edge_decoder(_pack_table(x_src), _pack_table(x_dst),
                         idx[0], idx[1])


# speedup vs baseline: 1.9561x; 1.5343x over previous
import functools

import jax
import jax.numpy as jnp
from jax import lax
from jax.experimental import pallas as pl
from jax.experimental.pallas import tpu as pltpu
from jax.experimental.pallas import tpu_sc as plsc

N_NODES_ = 10000
N_EDGES_ = 320000
D_ = 128

NC = 2
NS = 16
NW = NC * NS

E_PER_W = N_EDGES_ // NW
CHUNK = 80
N_CHUNKS = E_PER_W // CHUNK

_GATHER_DNUMS = lax.GatherDimensionNumbers(
    offset_dims=(), collapsed_slice_dims=(0,), start_index_map=(0,))


def _lane_perm(t, idx):
    return lax.gather(
        t, idx[:, None], _GATHER_DNUMS, slice_sizes=(1,),
        mode=lax.GatherScatterMode.PROMISE_IN_BOUNDS)


def _lane_sum(t):
    lane = lax.iota(jnp.int32, 16)
    for m in (8, 4, 2, 1):
        t = t + _lane_perm(t, lane ^ m)
    return t


def _dot_chunk(rows_a, rows_b, tmp_v, out_v, out_base):
    lane = lax.iota(jnp.int32, 16)

    @plsc.parallel_loop(0, CHUNK, 1, unroll=2)
    def _(e):
        parts = []
        for k in range(4):
            aw = rows_a[e, pl.ds(16 * k, 16)]
            bw = rows_b[e, pl.ds(16 * k, 16)]
            a_hi = lax.bitcast_convert_type(aw, jnp.float32)
            b_hi = lax.bitcast_convert_type(bw, jnp.float32)
            a_lo = lax.bitcast_convert_type(aw << 16, jnp.float32)
            b_lo = lax.bitcast_convert_type(bw << 16, jnp.float32)
            parts.append(a_hi * b_hi + a_lo * b_lo)
        s2 = [parts[0] + parts[1], parts[2] + parts[3]]
        t = _lane_sum(s2[0] + s2[1])
        tmp_v[pl.ds(e * 16, 16)] = t

    def compact_body(g, _):
        acc = jnp.zeros((16,), jnp.float32)
        for i in range(16):
            acc = jnp.where(lane == i, tmp_v[pl.ds((g * 16 + i) * 16, 16)],
                            acc)
        out_v[pl.ds(out_base + g * 16, 16)] = acc
        return 0

    lax.fori_loop(0, CHUNK // 16, compact_body, 0)


def _edge_decoder_kernel(x_src_hbm, x_dst_hbm, idx_src_hbm, idx_dst_hbm,
                         out_hbm, ia0, ib0, ia1, ib1,
                         rows_a0, rows_b0, rows_a1, rows_b1,
                         tmp_v, out_v, sh_a, sh_b, si0, si1, sa0, sb0, sa1,
                         sb1):
    sid = lax.axis_index("s")
    wid = sid * NC + lax.axis_index("c")
    base = wid * E_PER_W

    rows_share = N_NODES_ // NS
    pltpu.sync_copy(x_src_hbm.at[pl.ds(sid * rows_share, rows_share)],
                    sh_a.at[pl.ds(sid * rows_share, rows_share)])
    pltpu.sync_copy(x_dst_hbm.at[pl.ds(sid * rows_share, rows_share)],
                    sh_b.at[pl.ds(sid * rows_share, rows_share)])
    plsc.subcore_barrier()

    def issue_idx(c, ia, ib, si):
        off = base + c * CHUNK
        pltpu.async_copy(idx_src_hbm.at[pl.ds(off, CHUNK)], ia, si)
        pltpu.async_copy(idx_dst_hbm.at[pl.ds(off, CHUNK)], ib, si)

    def wait_idx(ia, ib, si):
        pltpu.make_async_copy(idx_src_hbm.at[pl.ds(0, CHUNK)], ia, si).wait()
        pltpu.make_async_copy(idx_dst_hbm.at[pl.ds(0, CHUNK)], ib, si).wait()

    def issue_rows(ia, ib, ra, rb, sa, sb):
        pltpu.async_copy(sh_a.at[ia], ra, sa)
        pltpu.async_copy(sh_b.at[ib], rb, sb)

    def wait_rows(ia, ib, ra, rb, sa, sb):
        pltpu.make_async_copy(sh_a.at[ia], ra, sa).wait()
        pltpu.make_async_copy(sh_b.at[ib], rb, sb).wait()

    pltpu.sync_copy(idx_src_hbm.at[pl.ds(base, CHUNK)], ia0)
    pltpu.sync_copy(idx_dst_hbm.at[pl.ds(base, CHUNK)], ib0)
    issue_rows(ia0, ib0, rows_a0, rows_b0, sa0, sb0)
    issue_idx(1, ia1, ib1, si1)

    def pair_body(g, _):
        c0 = 2 * g
        wait_idx(ia1, ib1, si1)
        issue_rows(ia1, ib1, rows_a1, rows_b1, sa1, sb1)
        wait_rows(ia0, ib0, rows_a0, rows_b0, sa0, sb0)
        issue_idx(c0 + 2, ia0, ib0, si0)
        _dot_chunk(rows_a0, rows_b0, tmp_v, out_v, c0 * CHUNK)
        wait_idx(ia0, ib0, si0)
        issue_rows(ia0, ib0, rows_a0, rows_b0, sa0, sb0)
        wait_rows(ia1, ib1, rows_a1, rows_b1, sa1, sb1)

        @pl.when(g < (N_CHUNKS - 3) // 2)
        def _():
            issue_idx(c0 + 3, ia1, ib1, si1)

        _dot_chunk(rows_a1, rows_b1, tmp_v, out_v, (c0 + 1) * CHUNK)
        return 0

    lax.fori_loop(0, (N_CHUNKS - 1) // 2, pair_body, 0)

    wait_rows(ia0, ib0, rows_a0, rows_b0, sa0, sb0)
    _dot_chunk(rows_a0, rows_b0, tmp_v, out_v, (N_CHUNKS - 1) * CHUNK)

    pltpu.sync_copy(out_v, out_hbm.at[pl.ds(base, E_PER_W)])


@jax.jit
def _edge_decoder(x_src, x_dst, idx_src, idx_dst):
    mesh = plsc.VectorSubcoreMesh(core_axis_name="c", subcore_axis_name="s")
    kfn = functools.partial(
        pl.kernel,
        mesh=mesh,
        compiler_params=pltpu.CompilerParams(use_tc_tiling_on_sc=False),
        out_type=jax.ShapeDtypeStruct((N_EDGES_,), jnp.float32),
        scratch_types=[
            pltpu.VMEM((CHUNK,), jnp.int32),
            pltpu.VMEM((CHUNK,), jnp.int32),
            pltpu.VMEM((CHUNK,), jnp.int32),
            pltpu.VMEM((CHUNK,), jnp.int32),
            pltpu.VMEM((CHUNK, D_ // 2), jnp.int32),
            pltpu.VMEM((CHUNK, D_ // 2), jnp.int32),
            pltpu.VMEM((CHUNK, D_ // 2), jnp.int32),
            pltpu.VMEM((CHUNK, D_ // 2), jnp.int32),
            pltpu.VMEM((CHUNK * 16,), jnp.float32),
            pltpu.VMEM((E_PER_W,), jnp.float32),
            pltpu.VMEM_SHARED((N_NODES_, D_ // 2), jnp.int32),
            pltpu.VMEM_SHARED((N_NODES_, D_ // 2), jnp.int32),
            pltpu.SemaphoreType.DMA,
            pltpu.SemaphoreType.DMA,
            pltpu.SemaphoreType.DMA,
            pltpu.SemaphoreType.DMA,
            pltpu.SemaphoreType.DMA,
            pltpu.SemaphoreType.DMA,
        ],
    )(_edge_decoder_kernel)
    return kfn(x_src, x_dst, idx_src, idx_dst)


def _pack_table(x):
    xb = x.astype(jnp.bfloat16).reshape(N_NODES_, D_ // 2, 2)
    return jax.lax.bitcast_convert_type(xb, jnp.int32)


def kernel(x_src, x_dst, edge_label_index):
    idx = edge_label_index.astype(jnp.int32)
    return _edge_decoder(_pack_table(x_src), _pack_table(x_dst),
                         idx[0], idx[1])
